# baseline (speedup 1.0000x reference)
import functools
import jax
import jax.numpy as jnp
from jax import lax
from jax.experimental import pallas as pl
from jax.experimental.pallas import tpu as pltpu
from jax.experimental.pallas import tpu_sc as plsc

N = 10000
E = 320000
NTILES = 32
EPT = E // NTILES
K = 80
NCHUNK = EPT // K
RPT = N // 16
RZ = 125


def _tc_mm2_body(x_ref, ps_ref, pd_ref, s_ref, d_ref):
    x = x_ref[...]
    s_ref[...] = jnp.dot(x, ps_ref[...], preferred_element_type=jnp.float32)
    d_ref[...] = jnp.dot(x, pd_ref[...], preferred_element_type=jnp.float32)


def _tc_mm2(x, ps, pd):
    n = x.shape[0]
    return pl.pallas_call(
        _tc_mm2_body,
        out_shape=(jax.ShapeDtypeStruct((n, ps.shape[1]), jnp.float32),
                   jax.ShapeDtypeStruct((n, pd.shape[1]), jnp.float32)),
    )(x, ps, pd)


def _tc_mid_body(a_ref, ps_ref, pd_ref, b_ref, s_ref, d_ref):
    a = a_ref[0] + a_ref[1]
    den = jnp.tile(a[:, 64:72], (1, 8))
    h = a[:, :64] / (den + 1e-16) + b_ref[...]
    h = jnp.where(h > 0, h, jnp.exp(h) - 1.0)
    s_ref[...] = jnp.dot(h, ps_ref[...], preferred_element_type=jnp.float32)
    d_ref[...] = jnp.dot(h, pd_ref[...], preferred_element_type=jnp.float32)


def _tc_mid(a1, ps, pd, b1p):
    return pl.pallas_call(
        _tc_mid_body,
        out_shape=(jax.ShapeDtypeStruct((N, ps.shape[1]), jnp.float32),
                   jax.ShapeDtypeStruct((N, pd.shape[1]), jnp.float32)),
    )(a1, ps, pd, b1p)


def _tc_fin_body(a_ref, b_ref, o_ref):
    a = a_ref[0] + a_ref[1]
    o_ref[...] = a[:, :40] / (a[:, 48:49] + 1e-16) + b_ref[...]


def _tc_fin(a2, b2):
    return pl.pallas_call(
        _tc_fin_body,
        out_shape=jax.ShapeDtypeStruct((N, 40), jnp.float32),
    )(a2, b2.reshape(1, 40))


def _make_sc_edge(W):
    nmsg = W // 16 - 1
    mesh = plsc.VectorSubcoreMesh(core_axis_name="c", subcore_axis_name="s")

    @functools.partial(
        pl.kernel, mesh=mesh,
        compiler_params=pltpu.CompilerParams(use_tc_tiling_on_sc=False),
        out_type=jax.ShapeDtypeStruct((2, N, W), jnp.float32),
        scratch_types=[
            pltpu.VMEM((NCHUNK, 1, K), jnp.int32),
            pltpu.VMEM((NCHUNK, 1, K), jnp.int32),
            pltpu.VMEM((2, K, W), jnp.float32),
            pltpu.VMEM((2, K, 16), jnp.float32),
            pltpu.VMEM((2, K, W), jnp.float32),
            pltpu.VMEM((RZ, W), jnp.float32),
            pltpu.VMEM_SHARED((N, W), jnp.float32),
            pltpu.SemaphoreType.DMA,
            pltpu.SemaphoreType.DMA,
            pltpu.SemaphoreType.DMA,
            pltpu.SemaphoreType.DMA,
            pltpu.SemaphoreType.DMA,
            pltpu.SemaphoreType.DMA,
        ],
    )
    def sc_edge(src_hbm, dst_hbm, s_hbm, d_hbm, out_hbm,
                sidx, didx, sbuf2, dbuf2, obuf2, zbuf, acc,
                semS0, semS1, semD0, semD1, semO0, semO1):
        semS = (semS0, semS1)
        semD = (semD0, semD1)
        semO = (semO0, semO1)
        sem1, sem2 = semS0, semD0
        c = lax.axis_index("c")
        s = lax.axis_index("s")
        tid = c * 16 + s

        cp1 = pltpu.async_copy(src_hbm.at[pl.ds(tid * NCHUNK, NCHUNK)], sidx,
                               sem1)
        cp2 = pltpu.async_copy(dst_hbm.at[pl.ds(tid * NCHUNK, NCHUNK)], didx,
                               sem2)

        zero = jnp.zeros((16,), jnp.float32)

        def zb(i, _):
            zbuf[i // (W // 16), pl.ds((i % (W // 16)) * 16, 16)] = zero
            return _
        lax.fori_loop(0, RZ * (W // 16), zb, None)

        def zacc(q, _):
            pltpu.sync_copy(zbuf, acc.at[pl.ds(s * RPT + q * RZ, RZ)])
            return _
        lax.fori_loop(0, RPT // RZ, zacc, None)
        cp1.wait()
        cp2.wait()
        plsc.subcore_barrier()

        def issue(b, g):
            return (pltpu.async_copy(s_hbm.at[sidx.at[g, 0]], sbuf2.at[b],
                                     semS[b]),
                    pltpu.async_copy(d_hbm.at[didx.at[g, 0]], dbuf2.at[b],
                                     semD[b]))

        def process(b, g):
            pltpu.make_async_copy(s_hbm.at[sidx.at[g, 0]], sbuf2.at[b],
                                  semS[b]).wait()
            pltpu.make_async_copy(d_hbm.at[didx.at[g, 0]], dbuf2.at[b],
                                  semD[b]).wait()
            sbuf = sbuf2.at[b]
            dbuf = dbuf2.at[b]
            obuf = obuf2.at[b]

            @pl.when(g >= 2)
            def _():
                pltpu.make_async_copy(obuf, acc.at[didx.at[g, 0]],
                                      semO[b]).wait()

            @plsc.parallel_loop(0, K, 1, unroll=8)
            def edge(e):
                aa = sbuf[e, pl.ds(W - 16, 16)] + dbuf[e, pl.ds(0, 16)]
                aa = jnp.maximum(aa, 0.2 * aa)
                w = jnp.exp(aa)
                obuf[e, pl.ds(W - 16, 16)] = w
                for j in range(nmsg):
                    obuf[e, pl.ds(16 * j, 16)] = sbuf[e, pl.ds(16 * j, 16)] * w

            pltpu.async_copy(obuf, acc.at[didx.at[g, 0]], semO[b], add=True)

        issue(0, 0)
        issue(1, 1)

        def chunk(p, _):
            g = 2 * p
            for b in range(2):
                process(b, g + b)
                issue(b, jnp.minimum(g + b + 2, NCHUNK - 1))
            return _
        lax.fori_loop(0, (NCHUNK - 1) // 2, chunk, None)
        process(0, NCHUNK - 1)
        pltpu.make_async_copy(s_hbm.at[sidx.at[0, 0]], sbuf2.at[1],
                              semS[1]).wait()
        pltpu.make_async_copy(d_hbm.at[didx.at[0, 0]], dbuf2.at[1],
                              semD[1]).wait()
        pltpu.make_async_copy(obuf2.at[0], acc.at[didx.at[0, 0]],
                              semO[0]).wait()
        pltpu.make_async_copy(obuf2.at[1], acc.at[didx.at[0, 0]],
                              semO[1]).wait()
        plsc.subcore_barrier()

        pltpu.sync_copy(acc.at[pl.ds(s * RPT, RPT)],
                        out_hbm.at[c, pl.ds(s * RPT, RPT)])

    return sc_edge


_sc_edge_80 = _make_sc_edge(80)
_sc_edge_64 = _make_sc_edge(64)


def kernel(x, edge_index, W1, as1, ad1, b1, W2, as2, ad2, b2):
    W1r = W1.reshape(128, 8, 8)
    P1h = jnp.transpose(W1r, (0, 2, 1)).reshape(128, 64)
    Wa1s = jnp.einsum('ihc,hc->ih', W1r, as1[0])
    Wa1d = jnp.einsum('ihc,hc->ih', W1r, ad1[0])
    P1S = jnp.concatenate([P1h, jnp.tile(Wa1s, (1, 2))], axis=1)
    P1D = jnp.tile(Wa1d, (1, 2))
    b1p = b1.reshape(8, 8).T.reshape(1, 64)

    idx64 = (jnp.arange(64) % 8) * 8 + jnp.arange(64) // 8
    W2p = W2[idx64]
    Wa2s = W2p @ as2[0, 0]
    Wa2d = W2p @ ad2[0, 0]
    P2S = jnp.concatenate([W2p, jnp.zeros((64, 8), jnp.float32),
                           jnp.tile(Wa2s[:, None], (1, 16))], axis=1)
    P2D = jnp.tile(Wa2d[:, None], (1, 16))

    src = edge_index[0].reshape(NTILES * NCHUNK, 1, K)
    dst = edge_index[1].reshape(NTILES * NCHUNK, 1, K)

    S1, D1 = _tc_mm2(x, P1S, P1D)
    A1 = _sc_edge_80(src, dst, S1, D1)
    S2, D2 = _tc_mid(A1, P2S, P2D, b1p)
    A2 = _sc_edge_64(src, dst, S2, D2)
    return _tc_fin(A2, b2)


# speedup vs baseline: 1.0926x; 1.0926x over previous
import functools
import jax
import jax.numpy as jnp
from jax import lax
from jax.experimental import pallas as pl
from jax.experimental.pallas import tpu as pltpu
from jax.experimental.pallas import tpu_sc as plsc

N = 10000
E = 320000
NTILES = 32
EPT = E // NTILES
K = 80
NCHUNK = EPT // K
RPT = N // 16
RZ = 125


def _tc_mm2_body(x_ref, ps_ref, pd_ref, s_ref, d_ref):
    x = x_ref[...]
    s_ref[...] = jnp.dot(x, ps_ref[...], preferred_element_type=jnp.float32)
    d_ref[...] = jnp.dot(x, pd_ref[...], preferred_element_type=jnp.float32)


def _tc_mm2(x, ps, pd):
    n = x.shape[0]
    return pl.pallas_call(
        _tc_mm2_body,
        out_shape=(jax.ShapeDtypeStruct((n, ps.shape[1]), jnp.float32),
                   jax.ShapeDtypeStruct((n, pd.shape[1]), jnp.float32)),
    )(x, ps, pd)


def _tc_mid_body(a_ref, ps_ref, pd_ref, b_ref, s_ref, d_ref):
    a = a_ref[0] + a_ref[1]
    den = jnp.tile(a[:, 64:72], (1, 8))
    h = a[:, :64] / (den + 1e-16) + b_ref[...]
    h = jnp.where(h > 0, h, jnp.exp(h) - 1.0)
    s_ref[...] = jnp.dot(h, ps_ref[...], preferred_element_type=jnp.float32)
    d_ref[...] = jnp.dot(h, pd_ref[...], preferred_element_type=jnp.float32)


def _tc_mid(a1, ps, pd, b1p):
    return pl.pallas_call(
        _tc_mid_body,
        out_shape=(jax.ShapeDtypeStruct((N, ps.shape[1]), jnp.float32),
                   jax.ShapeDtypeStruct((N, pd.shape[1]), jnp.float32)),
    )(a1, ps, pd, b1p)


def _tc_fin_body(a_ref, b_ref, o_ref):
    a = a_ref[0] + a_ref[1]
    o_ref[...] = a[:, :40] / (a[:, 48:49] + 1e-16) + b_ref[...]


def _tc_fin(a2, b2):
    return pl.pallas_call(
        _tc_fin_body,
        out_shape=jax.ShapeDtypeStruct((N, 40), jnp.float32),
    )(a2, b2.reshape(1, 40))


def _make_sc_edge(W):
    nmsg = W // 16 - 1
    mesh = plsc.VectorSubcoreMesh(core_axis_name="c", subcore_axis_name="s")

    @functools.partial(
        pl.kernel, mesh=mesh,
        compiler_params=pltpu.CompilerParams(use_tc_tiling_on_sc=False),
        out_type=jax.ShapeDtypeStruct((2, N, W), jnp.float32),
        scratch_types=[
            pltpu.VMEM((NCHUNK, 1, K), jnp.int32),
            pltpu.VMEM((NCHUNK, 1, K), jnp.int32),
            pltpu.VMEM((4, K, W), jnp.float32),
            pltpu.VMEM((4, K, 16), jnp.float32),
            pltpu.VMEM((2, K, W), jnp.float32),
            pltpu.VMEM((RZ, W), jnp.float32),
            pltpu.VMEM_SHARED((N, W), jnp.float32),
            [pltpu.SemaphoreType.DMA] * 4,
            [pltpu.SemaphoreType.DMA] * 4,
            [pltpu.SemaphoreType.DMA] * 2,
        ],
    )
    def sc_edge(src_hbm, dst_hbm, s_hbm, d_hbm, out_hbm,
                sidx, didx, sbuf2, dbuf2, obuf2, zbuf, acc,
                semS, semD, semO):
        sem1, sem2 = semS[0], semD[0]
        c = lax.axis_index("c")
        s = lax.axis_index("s")
        tid = c * 16 + s

        cp1 = pltpu.async_copy(src_hbm.at[pl.ds(tid * NCHUNK, NCHUNK)], sidx,
                               sem1)
        cp2 = pltpu.async_copy(dst_hbm.at[pl.ds(tid * NCHUNK, NCHUNK)], didx,
                               sem2)

        zero = jnp.zeros((16,), jnp.float32)

        def zb(i, _):
            zbuf[i // (W // 16), pl.ds((i % (W // 16)) * 16, 16)] = zero
            return _
        lax.fori_loop(0, RZ * (W // 16), zb, None)

        def zacc(q, _):
            pltpu.sync_copy(zbuf, acc.at[pl.ds(s * RPT + q * RZ, RZ)])
            return _
        lax.fori_loop(0, RPT // RZ, zacc, None)
        cp1.wait()
        cp2.wait()
        plsc.subcore_barrier()

        def issue(b, g):
            return (pltpu.async_copy(s_hbm.at[sidx.at[g, 0]], sbuf2.at[b],
                                     semS[b]),
                    pltpu.async_copy(d_hbm.at[didx.at[g, 0]], dbuf2.at[b],
                                     semD[b]))

        def process(b, g):
            pltpu.make_async_copy(s_hbm.at[sidx.at[g, 0]], sbuf2.at[b],
                                  semS[b]).wait()
            pltpu.make_async_copy(d_hbm.at[didx.at[g, 0]], dbuf2.at[b],
                                  semD[b]).wait()
            sbuf = sbuf2.at[b]
            dbuf = dbuf2.at[b]
            ob = b % 2
            obuf = obuf2.at[ob]

            @pl.when(g >= 2)
            def _():
                pltpu.make_async_copy(obuf, acc.at[didx.at[g, 0]],
                                      semO[ob]).wait()

            @plsc.parallel_loop(0, K, 1, unroll=8)
            def edge(e):
                aa = sbuf[e, pl.ds(W - 16, 16)] + dbuf[e, pl.ds(0, 16)]
                aa = jnp.maximum(aa, 0.2 * aa)
                w = jnp.exp(aa)
                obuf[e, pl.ds(W - 16, 16)] = w
                for j in range(nmsg):
                    obuf[e, pl.ds(16 * j, 16)] = sbuf[e, pl.ds(16 * j, 16)] * w

            pltpu.async_copy(obuf, acc.at[didx.at[g, 0]], semO[ob], add=True)

        for b in range(4):
            issue(b, b)

        def chunk(p, _):
            g = 4 * p
            for b in range(4):
                process(b, g + b)
                issue(b, jnp.minimum(g + b + 4, NCHUNK - 1))
            return _
        lax.fori_loop(0, (NCHUNK - 1) // 4, chunk, None)
        process(0, NCHUNK - 1)
        for b in range(1, 4):
            pltpu.make_async_copy(s_hbm.at[sidx.at[0, 0]], sbuf2.at[b],
                                  semS[b]).wait()
            pltpu.make_async_copy(d_hbm.at[didx.at[0, 0]], dbuf2.at[b],
                                  semD[b]).wait()
        pltpu.make_async_copy(obuf2.at[0], acc.at[didx.at[0, 0]],
                              semO[0]).wait()
        pltpu.make_async_copy(obuf2.at[1], acc.at[didx.at[0, 0]],
                              semO[1]).wait()
        plsc.subcore_barrier()

        pltpu.sync_copy(acc.at[pl.ds(s * RPT, RPT)],
                        out_hbm.at[c, pl.ds(s * RPT, RPT)])

    return sc_edge


_sc_edge_80 = _make_sc_edge(80)
_sc_edge_64 = _make_sc_edge(64)


def kernel(x, edge_index, W1, as1, ad1, b1, W2, as2, ad2, b2):
    W1r = W1.reshape(128, 8, 8)
    P1h = jnp.transpose(W1r, (0, 2, 1)).reshape(128, 64)
    Wa1s = jnp.einsum('ihc,hc->ih', W1r, as1[0])
    Wa1d = jnp.einsum('ihc,hc->ih', W1r, ad1[0])
    P1S = jnp.concatenate([P1h, jnp.tile(Wa1s, (1, 2))], axis=1)
    P1D = jnp.tile(Wa1d, (1, 2))
    b1p = b1.reshape(8, 8).T.reshape(1, 64)

    idx64 = (jnp.arange(64) % 8) * 8 + jnp.arange(64) // 8
    W2p = W2[idx64]
    Wa2s = W2p @ as2[0, 0]
    Wa2d = W2p @ ad2[0, 0]
    P2S = jnp.concatenate([W2p, jnp.zeros((64, 8), jnp.float32),
                           jnp.tile(Wa2s[:, None], (1, 16))], axis=1)
    P2D = jnp.tile(Wa2d[:, None], (1, 16))

    src = edge_index[0].reshape(NTILES * NCHUNK, 1, K)
    dst = edge_index[1].reshape(NTILES * NCHUNK, 1, K)

    S1, D1 = _tc_mm2(x, P1S, P1D)
    A1 = _sc_edge_80(src, dst, S1, D1)
    S2, D2 = _tc_mid(A1, P2S, P2D, b1p)
    A2 = _sc_edge_64(src, dst, S2, D2)
    return _tc_fin(A2, b2)
